# baseline (speedup 1.0000x reference)
import functools

import jax
import jax.numpy as jnp
from jax import lax
from jax.experimental import pallas as pl
from jax.experimental.pallas import tpu as pltpu
from jax.experimental.pallas import tpu_sc as plsc

EMBED = 128
N_ROWS = 4096 * 200
ROW_GROUPS = N_ROWS // EMBED
NC, NS = 2, 16
NW = NC * NS
PER_W = ROW_GROUPS // NW
HALF = PER_W // 2
CH = 2
NCHUNK = HALF // CH
NSTEP = NCHUNK // 2
CHUNK_WORDS = CH * EMBED * EMBED


def _tab_body(we_ref, wp_ref, wr_ref, out_ref):
    c = lax.broadcasted_iota(jnp.int32, (48, EMBED), 0)
    n = c % 5
    p = (c // 5) % 3
    r = c // 15
    t = jnp.zeros((48, EMBED), jnp.float32)
    for i in range(5):
        t = t + jnp.where(n == i, we_ref[i, :][None, :], 0.0)
    for i in range(3):
        t = t + jnp.where(p == i, wp_ref[i, :][None, :], 0.0)
    for i in range(3):
        t = t + jnp.where(r == i, wr_ref[i, :][None, :], 0.0)
    out_ref[...] = t


def _combined_table(We, Wpbs, Wrt):
    return pl.pallas_call(
        _tab_body,
        out_shape=jax.ShapeDtypeStruct((48, EMBED), jnp.float32),
    )(We, Wpbs, Wrt)


def _sc_embed(tab_hbm, xn_hbm, xp_hbm, xr_hbm, out_hbm,
              tab_v, xn_v, xp_v, xr_v, rows0, rows1, sw0, sw1):
    wid = lax.axis_index("s") * NC + lax.axis_index("c")
    rows = (rows0, rows1)
    sw = (sw0, sw1)
    lane = lax.broadcasted_iota(jnp.int32, (16,), 0)
    zero16 = jnp.zeros((16,), jnp.int32)

    pltpu.sync_copy(tab_hbm, tab_v)

    def write_chunk(i, slot, hbase):
        return pltpu.make_async_copy(
            rows[slot],
            out_hbm.at[pl.ds((hbase + i * CH) * EMBED * EMBED, CHUNK_WORDS)],
            sw[slot],
        )

    def compute_chunk(i, slot):
        def grp(jj, c0):
            r = i * CH + jj

            def sub(kk, c1):
                s = pl.ds(kk * 16, 16)
                v = xn_v[r, 0, s] + xp_v[r, 0, s] * 5 + xr_v[r, 0, s] * 15
                vbase = v * EMBED
                ribase = lane * EMBED + (jj * 8 + kk) * (16 * EMBED)

                def dblk(t2, c2):
                    for dd in range(16):
                        d = t2 * 16 + dd
                        cvec = d + zero16
                        vals = plsc.load_gather(tab_v, [vbase + cvec])
                        plsc.store_scatter(rows[slot], [ribase + cvec], vals)
                    return c2

                return lax.fori_loop(0, EMBED // 16, dblk, c1)

            return lax.fori_loop(0, 8, sub, c0)

        lax.fori_loop(0, CH, grp, 0)

    def half(h, carry):
        hbase = wid * PER_W + h * HALF
        pltpu.sync_copy(xn_hbm.at[pl.ds(hbase, HALF)], xn_v)
        pltpu.sync_copy(xp_hbm.at[pl.ds(hbase, HALF)], xp_v)
        pltpu.sync_copy(xr_hbm.at[pl.ds(hbase, HALF)], xr_v)

        def step(t, c):
            for s in range(2):
                i = 2 * t + s

                @pl.when(t > 0)
                def _drain():
                    write_chunk(i - 2, s, hbase).wait()

                compute_chunk(i, s)
                write_chunk(i, s, hbase).start()
            return c

        lax.fori_loop(0, NSTEP, step, 0)

        for s in range(2):
            write_chunk(NCHUNK - 2 + s, s, hbase).wait()
        return carry

    lax.fori_loop(0, 2, half, 0)


_sc_embed_call = functools.partial(
    pl.kernel,
    out_type=jax.ShapeDtypeStruct((N_ROWS * EMBED,), jnp.float32),
    mesh=plsc.VectorSubcoreMesh(core_axis_name="c", subcore_axis_name="s"),
    compiler_params=pltpu.CompilerParams(needs_layout_passes=False),
    scratch_types=[
        pltpu.VMEM((48 * EMBED,), jnp.float32),
        pltpu.VMEM((HALF, 1, EMBED), jnp.int32),
        pltpu.VMEM((HALF, 1, EMBED), jnp.int32),
        pltpu.VMEM((HALF, 1, EMBED), jnp.int32),
        pltpu.VMEM((CHUNK_WORDS,), jnp.float32),
        pltpu.VMEM((CHUNK_WORDS,), jnp.float32),
        pltpu.SemaphoreType.DMA,
        pltpu.SemaphoreType.DMA,
    ],
)(_sc_embed)


@jax.jit
def kernel(X_nucl, X_pbs, X_rt, We, Wpbs, Wrt):
    xn = X_nucl.astype(jnp.int32).reshape(ROW_GROUPS, 1, EMBED)
    xp = X_pbs.astype(jnp.int32).reshape(ROW_GROUPS, 1, EMBED)
    xr = X_rt.astype(jnp.int32).reshape(ROW_GROUPS, 1, EMBED)
    tab = _combined_table(We, Wpbs, Wrt).reshape(-1)
    out = _sc_embed_call(tab, xn, xp, xr)
    return out.reshape(X_nucl.shape[0], X_nucl.shape[1], EMBED)


# speedup vs baseline: 12.5043x; 12.5043x over previous
import functools

import jax
import jax.numpy as jnp
from jax import lax
from jax.experimental import pallas as pl
from jax.experimental.pallas import tpu as pltpu
from jax.experimental.pallas import tpu_sc as plsc

EMBED = 128
N_ROWS = 4096 * 200
ROW_GROUPS = N_ROWS // EMBED
NC, NS = 2, 16
NW = NC * NS
PER_W = ROW_GROUPS // NW
HALF = PER_W // 2
CH = 2
NCHUNK = HALF // CH
NSTEP = NCHUNK // 2


def _tab_body(we_ref, wp_ref, wr_ref, out_ref):
    c = lax.broadcasted_iota(jnp.int32, (48, EMBED), 0)
    n = c % 5
    p = (c // 5) % 3
    r = c // 15
    t = jnp.zeros((48, EMBED), jnp.float32)
    for i in range(5):
        t = t + jnp.where(n == i, we_ref[i, :][None, :], 0.0)
    for i in range(3):
        t = t + jnp.where(p == i, wp_ref[i, :][None, :], 0.0)
    for i in range(3):
        t = t + jnp.where(r == i, wr_ref[i, :][None, :], 0.0)
    out_ref[...] = t


def _combined_table(We, Wpbs, Wrt):
    return pl.pallas_call(
        _tab_body,
        out_shape=jax.ShapeDtypeStruct((48, EMBED), jnp.float32),
    )(We, Wpbs, Wrt)


def _sc_embed(tab_hbm, xn_hbm, xp_hbm, xr_hbm, out_hbm,
              tab_v, xn_v, xp_v, xr_v, rows0, rows1, sg0, sg1, sw0, sw1):
    wid = lax.axis_index("s") * NC + lax.axis_index("c")
    rows = (rows0, rows1)
    sg = (sg0, sg1)
    sw = (sw0, sw1)

    @pl.when(lax.axis_index("s") == 0)
    def _stage_tab():
        pltpu.sync_copy(tab_hbm, tab_v)

    plsc.subcore_barrier()

    def gather_chunk(i, slot):
        return [
            pltpu.make_async_copy(
                tab_v.at[xn_v.at[i * CH + j, 0]],
                rows[slot].at[pl.ds(j * EMBED, EMBED)],
                sg[slot],
            )
            for j in range(CH)
        ]

    def write_chunk(i, slot, hbase):
        return pltpu.make_async_copy(
            rows[slot],
            out_hbm.at[pl.ds((hbase + i * CH) * EMBED, CH * EMBED)],
            sw[slot],
        )

    def half(h, carry):
        hbase = wid * PER_W + h * HALF
        pltpu.sync_copy(xn_hbm.at[pl.ds(hbase, HALF)], xn_v)
        pltpu.sync_copy(xp_hbm.at[pl.ds(hbase, HALF)], xp_v)
        pltpu.sync_copy(xr_hbm.at[pl.ds(hbase, HALF)], xr_v)

        def combine(j, c):
            for k in range(EMBED // 16):
                s = pl.ds(k * 16, 16)
                xn_v[j, 0, s] = xn_v[j, 0, s] + xp_v[j, 0, s] * 5 + xr_v[j, 0, s] * 15
            return c

        lax.fori_loop(0, HALF, combine, 0)

        for s in range(2):
            for cp in gather_chunk(s, s):
                cp.start()

        def step(t, c):
            for s in range(2):
                i = 2 * t + s
                for cp in gather_chunk(i, s):
                    cp.wait()
                write_chunk(i, s, hbase).start()

            @pl.when(t < NSTEP - 1)
            def _prefetch():
                for s in range(2):
                    i = 2 * t + s
                    write_chunk(i, s, hbase).wait()
                    for cp in gather_chunk(i + 2, s):
                        cp.start()

            return c

        lax.fori_loop(0, NSTEP, step, 0)

        for s in range(2):
            write_chunk(NCHUNK - 2 + s, s, hbase).wait()
        return carry

    lax.fori_loop(0, 2, half, 0)


_sc_embed_call = functools.partial(
    pl.kernel,
    out_type=jax.ShapeDtypeStruct((N_ROWS, EMBED), jnp.float32),
    mesh=plsc.VectorSubcoreMesh(core_axis_name="c", subcore_axis_name="s"),
    scratch_types=[
        pltpu.VMEM_SHARED((48, EMBED), jnp.float32),
        pltpu.VMEM((HALF, 1, EMBED), jnp.int32),
        pltpu.VMEM((HALF, 1, EMBED), jnp.int32),
        pltpu.VMEM((HALF, 1, EMBED), jnp.int32),
        pltpu.VMEM((CH * EMBED, EMBED), jnp.float32),
        pltpu.VMEM((CH * EMBED, EMBED), jnp.float32),
        pltpu.SemaphoreType.DMA,
        pltpu.SemaphoreType.DMA,
        pltpu.SemaphoreType.DMA,
        pltpu.SemaphoreType.DMA,
    ],
)(_sc_embed)


@jax.jit
def kernel(X_nucl, X_pbs, X_rt, We, Wpbs, Wrt):
    xn = X_nucl.astype(jnp.int32).reshape(ROW_GROUPS, 1, EMBED)
    xp = X_pbs.astype(jnp.int32).reshape(ROW_GROUPS, 1, EMBED)
    xr = X_rt.astype(jnp.int32).reshape(ROW_GROUPS, 1, EMBED)
    tab = _combined_table(We, Wpbs, Wrt)
    out = _sc_embed_call(tab, xn, xp, xr)
    return out.reshape(X_nucl.shape[0], X_nucl.shape[1], EMBED)


# speedup vs baseline: 17.3094x; 1.3843x over previous
import functools

import jax
import jax.numpy as jnp
from jax import lax
from jax.experimental import pallas as pl
from jax.experimental.pallas import tpu as pltpu
from jax.experimental.pallas import tpu_sc as plsc

EMBED = 128
N_ROWS = 4096 * 200
ROW_GROUPS = N_ROWS // EMBED
NC, NS = 2, 16
NW = NC * NS
PER_W = ROW_GROUPS // NW
HALF = PER_W // 2
NSLOT = 4
NCHUNK = HALF
NSTEP = NCHUNK // NSLOT


def _tab_body(we_ref, wp_ref, wr_ref, out_ref):
    c = lax.broadcasted_iota(jnp.int32, (48, EMBED), 0)
    n = c % 5
    p = (c // 5) % 3
    r = c // 15
    t = jnp.zeros((48, EMBED), jnp.float32)
    for i in range(5):
        t = t + jnp.where(n == i, we_ref[i, :][None, :], 0.0)
    for i in range(3):
        t = t + jnp.where(p == i, wp_ref[i, :][None, :], 0.0)
    for i in range(3):
        t = t + jnp.where(r == i, wr_ref[i, :][None, :], 0.0)
    out_ref[...] = t


def _combined_table(We, Wpbs, Wrt):
    return pl.pallas_call(
        _tab_body,
        out_shape=jax.ShapeDtypeStruct((48, EMBED), jnp.float32),
    )(We, Wpbs, Wrt)


def _sc_embed(tab_hbm, xn_hbm, xp_hbm, xr_hbm, out_hbm,
              tab_v, xn_v, xp_v, xr_v,
              rows0, rows1, rows2, rows3,
              sg0, sg1, sg2, sg3, sw0, sw1, sw2, sw3):
    wid = lax.axis_index("s") * NC + lax.axis_index("c")
    rows = (rows0, rows1, rows2, rows3)
    sg = (sg0, sg1, sg2, sg3)
    sw = (sw0, sw1, sw2, sw3)

    @pl.when(lax.axis_index("s") == 0)
    def _stage_tab():
        pltpu.sync_copy(tab_hbm, tab_v)

    plsc.subcore_barrier()

    def gather_chunk(i, slot):
        return pltpu.make_async_copy(
            tab_v.at[xn_v.at[i, 0]], rows[slot], sg[slot]
        )

    def write_chunk(i, slot, hbase):
        return pltpu.make_async_copy(
            rows[slot],
            out_hbm.at[pl.ds((hbase + i) * EMBED, EMBED)],
            sw[slot],
        )

    def half(h, carry):
        hbase = wid * PER_W + h * HALF
        pltpu.sync_copy(xn_hbm.at[pl.ds(hbase, HALF)], xn_v)
        pltpu.sync_copy(xp_hbm.at[pl.ds(hbase, HALF)], xp_v)
        pltpu.sync_copy(xr_hbm.at[pl.ds(hbase, HALF)], xr_v)

        def combine(j, c):
            for k in range(EMBED // 16):
                s = pl.ds(k * 16, 16)
                xn_v[j, 0, s] = xn_v[j, 0, s] + xp_v[j, 0, s] * 5 + xr_v[j, 0, s] * 15
            return c

        lax.fori_loop(0, HALF, combine, 0)

        for s in range(NSLOT):
            gather_chunk(s, s).start()

        def step(t, c):
            for s in range(NSLOT):
                i = NSLOT * t + s
                gather_chunk(i, s).wait()
                write_chunk(i, s, hbase).start()

            @pl.when(t < NSTEP - 1)
            def _prefetch():
                for s in range(NSLOT):
                    i = NSLOT * t + s
                    write_chunk(i, s, hbase).wait()
                    gather_chunk(i + NSLOT, s).start()

            return c

        lax.fori_loop(0, NSTEP, step, 0)

        for s in range(NSLOT):
            write_chunk(NCHUNK - NSLOT + s, s, hbase).wait()
        return carry

    lax.fori_loop(0, 2, half, 0)


_sc_embed_call = functools.partial(
    pl.kernel,
    out_type=jax.ShapeDtypeStruct((N_ROWS, EMBED), jnp.float32),
    mesh=plsc.VectorSubcoreMesh(core_axis_name="c", subcore_axis_name="s"),
    scratch_types=[
        pltpu.VMEM_SHARED((48, EMBED), jnp.float32),
        pltpu.VMEM((HALF, 1, EMBED), jnp.int32),
        pltpu.VMEM((HALF, 1, EMBED), jnp.int32),
        pltpu.VMEM((HALF, 1, EMBED), jnp.int32),
        pltpu.VMEM((EMBED, EMBED), jnp.float32),
        pltpu.VMEM((EMBED, EMBED), jnp.float32),
        pltpu.VMEM((EMBED, EMBED), jnp.float32),
        pltpu.VMEM((EMBED, EMBED), jnp.float32),
        pltpu.SemaphoreType.DMA,
        pltpu.SemaphoreType.DMA,
        pltpu.SemaphoreType.DMA,
        pltpu.SemaphoreType.DMA,
        pltpu.SemaphoreType.DMA,
        pltpu.SemaphoreType.DMA,
        pltpu.SemaphoreType.DMA,
        pltpu.SemaphoreType.DMA,
    ],
)(_sc_embed)


@jax.jit
def kernel(X_nucl, X_pbs, X_rt, We, Wpbs, Wrt):
    xn = X_nucl.astype(jnp.int32).reshape(ROW_GROUPS, 1, EMBED)
    xp = X_pbs.astype(jnp.int32).reshape(ROW_GROUPS, 1, EMBED)
    xr = X_rt.astype(jnp.int32).reshape(ROW_GROUPS, 1, EMBED)
    tab = _combined_table(We, Wpbs, Wrt)
    out = _sc_embed_call(tab, xn, xp, xr)
    return out.reshape(X_nucl.shape[0], X_nucl.shape[1], EMBED)


# speedup vs baseline: 17.8301x; 1.0301x over previous
import functools

import jax
import jax.numpy as jnp
from jax import lax
from jax.experimental import pallas as pl
from jax.experimental.pallas import tpu as pltpu
from jax.experimental.pallas import tpu_sc as plsc

EMBED = 128
N_ROWS = 4096 * 200
ROW_GROUPS = N_ROWS // EMBED
NC, NS = 2, 16
NW = NC * NS
PER_W = ROW_GROUPS // NW
QTR = PER_W // 4
NSLOT = 5
NSTEP = PER_W // NSLOT
SPQ = QTR // (PER_W // NSLOT // 4)


def _tab_body(we_ref, wp_ref, wr_ref, out_ref):
    c = lax.broadcasted_iota(jnp.int32, (48, EMBED), 0)
    n = c % 5
    p = (c // 5) % 3
    r = c // 15
    t = jnp.zeros((48, EMBED), jnp.float32)
    for i in range(5):
        t = t + jnp.where(n == i, we_ref[i, :][None, :], 0.0)
    for i in range(3):
        t = t + jnp.where(p == i, wp_ref[i, :][None, :], 0.0)
    for i in range(3):
        t = t + jnp.where(r == i, wr_ref[i, :][None, :], 0.0)
    out_ref[...] = t


def _combined_table(We, Wpbs, Wrt):
    return pl.pallas_call(
        _tab_body,
        out_shape=jax.ShapeDtypeStruct((48, EMBED), jnp.float32),
    )(We, Wpbs, Wrt)


def _sc_embed(tab_hbm, xn_hbm, xp_hbm, xr_hbm, out_hbm,
              tab_sh, xc_v, xn_v, xp_v, xr_v,
              rows0, rows1, rows2, rows3, rows4,
              sg0, sg1, sg2, sg3, sg4, sw0, sw1, sw2, sw3, sw4, si):
    wid = lax.axis_index("s") * NC + lax.axis_index("c")
    base = wid * PER_W
    rows = (rows0, rows1, rows2, rows3, rows4)
    sg = (sg0, sg1, sg2, sg3, sg4)
    sw = (sw0, sw1, sw2, sw3, sw4)

    @pl.when(lax.axis_index("s") == 0)
    def _stage_tab():
        pltpu.sync_copy(tab_hbm, tab_sh)

    plsc.subcore_barrier()

    def slab_copies(q):
        sl = pl.ds(base + q * QTR, QTR)
        return [
            pltpu.make_async_copy(xn_hbm.at[sl], xn_v, si),
            pltpu.make_async_copy(xp_hbm.at[sl], xp_v, si),
            pltpu.make_async_copy(xr_hbm.at[sl], xr_v, si),
        ]

    def combine_one(src, dst):
        for k in range(EMBED // 16):
            s = pl.ds(k * 16, 16)
            xc_v[dst, 0, s] = (
                xn_v[src, 0, s] + xp_v[src, 0, s] * 5 + xr_v[src, 0, s] * 15
            )

    def gather_chunk(i, slot):
        return pltpu.make_async_copy(tab_sh.at[xc_v.at[i, 0]], rows[slot], sg[slot])

    def write_chunk(i, slot):
        return pltpu.make_async_copy(
            rows[slot], out_hbm.at[pl.ds((base + i) * EMBED, EMBED)], sw[slot]
        )

    for cp in slab_copies(0):
        cp.start()
    for cp in slab_copies(0):
        cp.wait()

    def comb0(j, c):
        combine_one(j, j)
        return c

    lax.fori_loop(0, QTR, comb0, 0)

    for cp in slab_copies(1):
        cp.start()

    for s in range(NSLOT):
        gather_chunk(s, s).start()

    steps_per_q = NSTEP // 4

    for q in range(4):
        def step(tt, c, q=q):
            t = q * steps_per_q + tt

            if q < 3:
                @pl.when(tt == 0)
                def _slab_arrived():
                    for cp in slab_copies(q + 1):
                        cp.wait()

                def one(k, cc):
                    combine_one(SPQ * tt + k, SPQ * t + QTR + k)
                    return cc

                lax.fori_loop(0, SPQ, one, 0)

                if q < 2:
                    @pl.when(tt == steps_per_q - 1)
                    def _slab_next():
                        for cp in slab_copies(q + 2):
                            cp.start()

            for s in range(NSLOT):
                i = NSLOT * t + s
                gather_chunk(i, s).wait()
                write_chunk(i, s).start()

            if q < 3:
                def _prefetch_body():
                    for s in range(NSLOT):
                        i = NSLOT * t + s
                        write_chunk(i, s).wait()
                        gather_chunk(i + NSLOT, s).start()
                _prefetch_body()
            else:
                @pl.when(tt < steps_per_q - 1)
                def _prefetch():
                    for s in range(NSLOT):
                        i = NSLOT * t + s
                        write_chunk(i, s).wait()
                        gather_chunk(i + NSLOT, s).start()

            return c

        lax.fori_loop(0, steps_per_q, step, 0)

    for s in range(NSLOT):
        write_chunk(PER_W - NSLOT + s, s).wait()


_sc_embed_call = functools.partial(
    pl.kernel,
    out_type=jax.ShapeDtypeStruct((N_ROWS, EMBED), jnp.float32),
    mesh=plsc.VectorSubcoreMesh(core_axis_name="c", subcore_axis_name="s"),
    scratch_types=[
        pltpu.VMEM_SHARED((48, EMBED), jnp.float32),
        pltpu.VMEM((PER_W, 1, EMBED), jnp.int32),
        pltpu.VMEM((QTR, 1, EMBED), jnp.int32),
        pltpu.VMEM((QTR, 1, EMBED), jnp.int32),
        pltpu.VMEM((QTR, 1, EMBED), jnp.int32),
        pltpu.VMEM((EMBED, EMBED), jnp.float32),
        pltpu.VMEM((EMBED, EMBED), jnp.float32),
        pltpu.VMEM((EMBED, EMBED), jnp.float32),
        pltpu.VMEM((EMBED, EMBED), jnp.float32),
        pltpu.VMEM((EMBED, EMBED), jnp.float32),
        pltpu.SemaphoreType.DMA,
        pltpu.SemaphoreType.DMA,
        pltpu.SemaphoreType.DMA,
        pltpu.SemaphoreType.DMA,
        pltpu.SemaphoreType.DMA,
        pltpu.SemaphoreType.DMA,
        pltpu.SemaphoreType.DMA,
        pltpu.SemaphoreType.DMA,
        pltpu.SemaphoreType.DMA,
        pltpu.SemaphoreType.DMA,
        pltpu.SemaphoreType.DMA,
    ],
)(_sc_embed)


@jax.jit
def kernel(X_nucl, X_pbs, X_rt, We, Wpbs, Wrt):
    xn = X_nucl.astype(jnp.int32).reshape(ROW_GROUPS, 1, EMBED)
    xp = X_pbs.astype(jnp.int32).reshape(ROW_GROUPS, 1, EMBED)
    xr = X_rt.astype(jnp.int32).reshape(ROW_GROUPS, 1, EMBED)
    tab = _combined_table(We, Wpbs, Wrt)
    out = _sc_embed_call(tab, xn, xp, xr)
    return out.reshape(X_nucl.shape[0], X_nucl.shape[1], EMBED)
